# baseline (speedup 1.0000x reference)
import functools

import jax
import jax.numpy as jnp
from jax import lax
from jax.experimental import pallas as pl
from jax.experimental.pallas import tpu as pltpu
from jax.experimental.pallas import tpu_sc as plsc

NUM_BUCKETS = 100000
VOCAB = 1000
SOFTCAP = 30.0
N_TOK = 1024 * 50
SEQ = 50

NC, NS, L = 2, 16, 16
NW = NC * NS
ROWS_PER_W = N_TOK // NW
CHUNK = 32
N_CHUNKS = ROWS_PER_W // CHUNK


def _sc_body(ids_hbm, tgt_hbm, w_hbm, sums_hbm, cts_hbm,
             ids_v, tgt_v, idx_v, rows_v, sums_v, cts_v, sem):
    i32 = jnp.int32
    wid = lax.axis_index("s") * NC + lax.axis_index("c")
    base = wid * ROWS_PER_W

    pltpu.sync_copy(ids_hbm.at[pl.ds(base, ROWS_PER_W)], ids_v)
    pltpu.sync_copy(tgt_hbm.at[pl.ds(base, ROWS_PER_W)], tgt_v)

    iota = lax.broadcasted_iota(jnp.int32, (L,), 0)

    def hash_block(i, _):
        l0 = i * L
        prev1 = ids_v[pl.ds(l0, L)]
        lidx = l0 + iota
        prev2 = plsc.load_gather(ids_v, [jnp.maximum(lidx - 1, 0)])
        s_pos = (base + lidx) % SEQ
        prev2 = jnp.where(s_pos == 0, 0, prev2)
        h = ((prev2 * 1000003) ^ (prev1 * 92821)) % NUM_BUCKETS
        idx_v[pl.ds(l0, L)] = h
        return 0

    lax.fori_loop(0, ROWS_PER_W // L, hash_block, 0, unroll=4)

    inv15 = jnp.float32(1.0 / 15.0)

    def softcap(x):
        u = jnp.exp(x * inv15)
        return jnp.float32(30.0) - jnp.float32(60.0) / (u + jnp.float32(1.0))

    def start_gather(k, slot):
        return pltpu.make_async_copy(
            w_hbm.at[idx_v.at[pl.ds(k * CHUNK, CHUNK)]],
            rows_v.at[slot], sem)

    start_gather(0, 0).start()

    def chunk_body(k, _):
        slot = lax.rem(k, 2)

        @pl.when(k + 1 < N_CHUNKS)
        def _():
            start_gather(k + 1, 1 - slot).start()

        pltpu.make_async_copy(
            w_hbm.at[idx_v.at[pl.ds(k * CHUNK, CHUNK)]],
            rows_v.at[slot], sem).wait()

        for g in range(CHUNK // L):
            rows16 = g * L + iota
            lrow = k * CHUNK + g * L

            def col_body(p, acc):
                col = jnp.broadcast_to(p, (L,))
                x = plsc.load_gather(rows_v.at[slot], [rows16, col])
                return acc + jnp.exp(softcap(x))

            acc = lax.fori_loop(0, VOCAB, col_body,
                                jnp.zeros((L,), jnp.float32), unroll=4)

            tg16 = tgt_v[pl.ds(lrow, L)]
            ct16 = softcap(plsc.load_gather(rows_v.at[slot], [rows16, tg16]))

            m0 = ((base + lrow + iota) % SEQ) == 0
            acc = jnp.where(m0, jnp.float32(float(VOCAB)), acc)
            ct16 = jnp.where(m0, jnp.float32(0.0), ct16)
            sums_v[pl.ds(lrow, L)] = acc
            cts_v[pl.ds(lrow, L)] = ct16
        return 0

    lax.fori_loop(0, N_CHUNKS, chunk_body, 0)

    pltpu.sync_copy(sums_v, sums_hbm.at[pl.ds(base, ROWS_PER_W)])
    pltpu.sync_copy(cts_v, cts_hbm.at[pl.ds(base, ROWS_PER_W)])


@jax.jit
def _sc_gather_ce(ids_i32, tgt_i32, W):
    with jax.enable_x64(False):
        return _sc_gather_ce_x32(ids_i32, tgt_i32, W)


def _sc_gather_ce_x32(ids_i32, tgt_i32, W):
    mesh = plsc.VectorSubcoreMesh(core_axis_name="c", subcore_axis_name="s")
    f = pl.kernel(
        _sc_body,
        out_type=(
            jax.ShapeDtypeStruct((N_TOK,), jnp.float32),
            jax.ShapeDtypeStruct((N_TOK,), jnp.float32),
        ),
        mesh=mesh,
        scratch_types=[
            pltpu.VMEM((ROWS_PER_W,), jnp.int32),
            pltpu.VMEM((ROWS_PER_W,), jnp.int32),
            pltpu.VMEM((ROWS_PER_W,), jnp.int32),
            pltpu.VMEM((2, CHUNK, VOCAB), jnp.float32),
            pltpu.VMEM((ROWS_PER_W,), jnp.float32),
            pltpu.VMEM((ROWS_PER_W,), jnp.float32),
            pltpu.SemaphoreType.DMA,
        ],
        compiler_params=pltpu.CompilerParams(
            needs_layout_passes=False, use_tc_tiling_on_sc=False),
    )
    return f(ids_i32, tgt_i32, W)


def _reduce_body(s_ref, c_ref, o_ref):
    nll = jnp.log(s_ref[...]) - c_ref[...]
    o_ref[0, 0] = jnp.sum(nll) * jnp.float32(1.0 / N_TOK)


@jax.jit
def _tc_reduce(sums, cts):
    out = pl.pallas_call(
        _reduce_body,
        out_shape=jax.ShapeDtypeStruct((1, 1), jnp.float32),
        out_specs=pl.BlockSpec(memory_space=pltpu.SMEM),
    )(sums.reshape(400, 128), cts.reshape(400, 128))
    return out[0, 0]


def kernel(input_ids, target_ids, W):
    ids = input_ids.reshape(-1).astype(jnp.int32)
    tgt = target_ids.reshape(-1).astype(jnp.int32)
    w_flat = lax.optimization_barrier(W.reshape(-1))
    w_lin = w_flat.reshape(NUM_BUCKETS, VOCAB)
    sums, cts = _sc_gather_ce(ids, tgt, w_lin)
    return _tc_reduce(sums, cts)


# speedup vs baseline: 1.6024x; 1.6024x over previous
import functools

import jax
import jax.numpy as jnp
from jax import lax
from jax.experimental import pallas as pl
from jax.experimental.pallas import tpu as pltpu
from jax.experimental.pallas import tpu_sc as plsc

NUM_BUCKETS = 100000
VOCAB = 1000
SOFTCAP = 30.0
N_TOK = 1024 * 50
SEQ = 50

NC, NS, L = 2, 16, 16
NW = NC * NS
ROWS_PER_W = N_TOK // NW
CHUNK = 32
N_CHUNKS = ROWS_PER_W // CHUNK


def _sc_body(ids_hbm, tgt_hbm, w_hbm, sums_hbm, cts_hbm,
             ids_v, tgt_v, idx_v, rows_v, sums_v, cts_v, sem):
    i32 = jnp.int32
    wid = lax.axis_index("s") * NC + lax.axis_index("c")
    base = wid * ROWS_PER_W

    pltpu.sync_copy(ids_hbm.at[pl.ds(base, ROWS_PER_W)], ids_v)
    pltpu.sync_copy(tgt_hbm.at[pl.ds(base, ROWS_PER_W)], tgt_v)

    iota = lax.broadcasted_iota(jnp.int32, (L,), 0)

    def hash_block(i, _):
        l0 = i * L
        prev1 = ids_v[pl.ds(l0, L)]
        lidx = l0 + iota
        prev2 = plsc.load_gather(ids_v, [jnp.maximum(lidx - 1, 0)])
        s_pos = (base + lidx) % SEQ
        prev2 = jnp.where(s_pos == 0, 0, prev2)
        h = ((prev2 * 1000003) ^ (prev1 * 92821)) % NUM_BUCKETS
        idx_v[pl.ds(l0, L)] = h
        return 0

    lax.fori_loop(0, ROWS_PER_W // L, hash_block, 0, unroll=4)

    inv15 = jnp.float32(1.0 / 15.0)

    def softcap(x):
        u = jnp.exp(x * inv15)
        return jnp.float32(30.0) - jnp.float32(60.0) / (u + jnp.float32(1.0))

    def start_gather(k, slot):
        return pltpu.make_async_copy(
            w_hbm.at[idx_v.at[pl.ds(k * CHUNK, CHUNK)]],
            rows_v.at[slot], sem)

    start_gather(0, 0).start()

    def chunk_body(k, _):
        slot = lax.rem(k, 2)

        @pl.when(k + 1 < N_CHUNKS)
        def _():
            start_gather(k + 1, 1 - slot).start()

        pltpu.make_async_copy(
            w_hbm.at[idx_v.at[pl.ds(k * CHUNK, CHUNK)]],
            rows_v.at[slot], sem).wait()

        for g in range(CHUNK // L):
            rows16 = g * L + iota
            lrow = k * CHUNK + g * L

            acc = jnp.zeros((L,), jnp.float32)
            for s in range(8):
                s16 = jnp.full((L,), s, jnp.int32)

                def col_body(p, a, s16=s16):
                    col = jnp.broadcast_to(p, (L,))
                    x = plsc.load_gather(rows_v.at[slot],
                                         [rows16, s16, col])
                    return a + jnp.exp(softcap(x))

                acc = lax.fori_loop(0, 128, col_body, acc, unroll=4)

            tg16 = tgt_v[pl.ds(lrow, L)]
            ct16 = softcap(plsc.load_gather(
                rows_v.at[slot],
                [rows16, jnp.right_shift(tg16, 7),
                 jnp.bitwise_and(tg16, 127)]))

            m0 = ((base + lrow + iota) % SEQ) == 0
            acc = jnp.where(m0, jnp.float32(1024.0), acc)
            ct16 = jnp.where(m0, jnp.float32(0.0), ct16)
            sums_v[pl.ds(lrow, L)] = acc
            cts_v[pl.ds(lrow, L)] = ct16
        return 0

    lax.fori_loop(0, N_CHUNKS, chunk_body, 0)

    pltpu.sync_copy(sums_v, sums_hbm.at[pl.ds(base, ROWS_PER_W)])
    pltpu.sync_copy(cts_v, cts_hbm.at[pl.ds(base, ROWS_PER_W)])


@jax.jit
def _sc_gather_ce(ids_i32, tgt_i32, W):
    with jax.enable_x64(False):
        return _sc_gather_ce_x32(ids_i32, tgt_i32, W)


def _sc_gather_ce_x32(ids_i32, tgt_i32, W):
    mesh = plsc.VectorSubcoreMesh(core_axis_name="c", subcore_axis_name="s")
    f = pl.kernel(
        _sc_body,
        out_type=(
            jax.ShapeDtypeStruct((N_TOK,), jnp.float32),
            jax.ShapeDtypeStruct((N_TOK,), jnp.float32),
        ),
        mesh=mesh,
        scratch_types=[
            pltpu.VMEM((ROWS_PER_W,), jnp.int32),
            pltpu.VMEM((ROWS_PER_W,), jnp.int32),
            pltpu.VMEM((ROWS_PER_W,), jnp.int32),
            pltpu.VMEM((2, CHUNK, 8, 128), jnp.float32),
            pltpu.VMEM((ROWS_PER_W,), jnp.float32),
            pltpu.VMEM((ROWS_PER_W,), jnp.float32),
            pltpu.SemaphoreType.DMA,
        ],
        compiler_params=pltpu.CompilerParams(needs_layout_passes=False),
    )
    return f(ids_i32, tgt_i32, W)


def _reduce_body(s_ref, c_ref, o_ref):
    nll = jnp.log(s_ref[...] - jnp.float32(24.0)) - c_ref[...]
    o_ref[0, 0] = jnp.sum(nll) * jnp.float32(1.0 / N_TOK)


@jax.jit
def _tc_reduce(sums, cts):
    out = pl.pallas_call(
        _reduce_body,
        out_shape=jax.ShapeDtypeStruct((1, 1), jnp.float32),
        out_specs=pl.BlockSpec(memory_space=pltpu.SMEM),
    )(sums.reshape(400, 128), cts.reshape(400, 128))
    return out[0, 0]


def kernel(input_ids, target_ids, W):
    ids = input_ids.reshape(-1).astype(jnp.int32)
    tgt = target_ids.reshape(-1).astype(jnp.int32)
    w3 = jnp.pad(W, ((0, 0), (0, 24))).reshape(NUM_BUCKETS, 8, 128)
    sums, cts = _sc_gather_ce(ids, tgt, w3)
    return _tc_reduce(sums, cts)


# speedup vs baseline: 2.8352x; 1.7694x over previous
import functools

import jax
import jax.numpy as jnp
from jax import lax
from jax.experimental import pallas as pl
from jax.experimental.pallas import tpu as pltpu
from jax.experimental.pallas import tpu_sc as plsc

NUM_BUCKETS = 100000
VOCAB = 1000
SOFTCAP = 30.0
N_TOK = 1024 * 50
SEQ = 50

NC, NS, L = 2, 16, 16
NW = NC * NS
ROWS_PER_W = N_TOK // NW
CHUNK = 32
N_CHUNKS = ROWS_PER_W // CHUNK


def _sc_body(ids_hbm, tgt_hbm, w_hbm, sums_hbm, cts_hbm,
             ids_v, tgt_v, idx_v, rows_v, sums_v, cts_v, sem):
    i32 = jnp.int32
    wid = lax.axis_index("s") * NC + lax.axis_index("c")
    base = wid * ROWS_PER_W

    pltpu.sync_copy(ids_hbm.at[pl.ds(base, ROWS_PER_W)], ids_v)
    pltpu.sync_copy(tgt_hbm.at[pl.ds(base, ROWS_PER_W)], tgt_v)

    iota = lax.broadcasted_iota(jnp.int32, (L,), 0)

    def hash_block(i, _):
        l0 = i * L
        prev1 = ids_v[pl.ds(l0, L)]
        lidx = l0 + iota
        prev2 = plsc.load_gather(ids_v, [jnp.maximum(lidx - 1, 0)])
        s_pos = (base + lidx) % SEQ
        prev2 = jnp.where(s_pos == 0, 0, prev2)
        h = ((prev2 * 1000003) ^ (prev1 * 92821)) % NUM_BUCKETS
        idx_v[pl.ds(l0, L)] = h
        return 0

    lax.fori_loop(0, ROWS_PER_W // L, hash_block, 0, unroll=4)

    inv15 = jnp.float32(1.0 / 15.0)

    def softcap(x):
        u = jnp.exp(x * inv15)
        return jnp.float32(30.0) - jnp.float32(60.0) / (u + jnp.float32(1.0))

    def start_gather(k, slot):
        return pltpu.make_async_copy(
            w_hbm.at[idx_v.at[pl.ds(k * CHUNK, CHUNK)]],
            rows_v.at[slot], sem)

    start_gather(0, 0).start()

    def chunk_body(k, _):
        slot = lax.rem(k, 2)

        @pl.when(k + 1 < N_CHUNKS)
        def _():
            start_gather(k + 1, 1 - slot).start()

        pltpu.make_async_copy(
            w_hbm.at[idx_v.at[pl.ds(k * CHUNK, CHUNK)]],
            rows_v.at[slot], sem).wait()

        for g in range(CHUNK // L):
            rows16 = g * L + iota
            lrow = k * CHUNK + g * L

            tg16 = tgt_v[pl.ds(lrow, L)]
            ct16 = softcap(plsc.load_gather(
                rows_v.at[slot],
                [rows16, jnp.right_shift(tg16, 7),
                 jnp.bitwise_and(tg16, 127)]))
            cts_v[pl.ds(lrow, L)] = ct16

            def tok_body(j, sums16):
                t = g * L + j
                acc = jnp.zeros((L,), jnp.float32)
                for s in range(8):
                    nh = 8 if s < 7 else 7
                    for h in range(nh):
                        x = rows_v[slot, t, s, pl.ds(h * L, L)]
                        e = jnp.exp(softcap(x))
                        if s == 7 and h == 6:
                            e = jnp.where(iota < 8, e, jnp.float32(0.0))
                        acc = acc + e
                total = jnp.sum(acc)
                return jnp.where(iota == j, total, sums16)

            sums16 = lax.fori_loop(0, L, tok_body,
                                   jnp.zeros((L,), jnp.float32))
            sums_v[pl.ds(lrow, L)] = sums16
        return 0

    lax.fori_loop(0, N_CHUNKS, chunk_body, 0)

    pltpu.sync_copy(sums_v, sums_hbm.at[pl.ds(base, ROWS_PER_W)])
    pltpu.sync_copy(cts_v, cts_hbm.at[pl.ds(base, ROWS_PER_W)])


@jax.jit
def _sc_gather_ce(ids_i32, tgt_i32, W):
    with jax.enable_x64(False):
        return _sc_gather_ce_x32(ids_i32, tgt_i32, W)


def _sc_gather_ce_x32(ids_i32, tgt_i32, W):
    mesh = plsc.VectorSubcoreMesh(core_axis_name="c", subcore_axis_name="s")
    f = pl.kernel(
        _sc_body,
        out_type=(
            jax.ShapeDtypeStruct((N_TOK,), jnp.float32),
            jax.ShapeDtypeStruct((N_TOK,), jnp.float32),
        ),
        mesh=mesh,
        scratch_types=[
            pltpu.VMEM((ROWS_PER_W,), jnp.int32),
            pltpu.VMEM((ROWS_PER_W,), jnp.int32),
            pltpu.VMEM((ROWS_PER_W,), jnp.int32),
            pltpu.VMEM((2, CHUNK, 8, 128), jnp.float32),
            pltpu.VMEM((ROWS_PER_W,), jnp.float32),
            pltpu.VMEM((ROWS_PER_W,), jnp.float32),
            pltpu.SemaphoreType.DMA,
        ],
        compiler_params=pltpu.CompilerParams(needs_layout_passes=False),
    )
    return f(ids_i32, tgt_i32, W)


def _reduce_body(s_ref, c_ref, o_ref):
    idx = (lax.broadcasted_iota(jnp.int32, (400, 128), 0) * 128
           + lax.broadcasted_iota(jnp.int32, (400, 128), 1))
    m0 = (idx % SEQ) == 0
    nll = jnp.where(m0, jnp.float32(6.907755278982137),
                    jnp.log(s_ref[...]) - c_ref[...])
    o_ref[0, 0] = jnp.sum(nll) * jnp.float32(1.0 / N_TOK)


@jax.jit
def _tc_reduce(sums, cts):
    out = pl.pallas_call(
        _reduce_body,
        out_shape=jax.ShapeDtypeStruct((1, 1), jnp.float32),
        out_specs=pl.BlockSpec(memory_space=pltpu.SMEM),
    )(sums.reshape(400, 128), cts.reshape(400, 128))
    return out[0, 0]


def kernel(input_ids, target_ids, W):
    ids = input_ids.reshape(-1).astype(jnp.int32)
    tgt = target_ids.reshape(-1).astype(jnp.int32)
    w3 = jnp.pad(W, ((0, 0), (0, 24))).reshape(NUM_BUCKETS, 8, 128)
    sums, cts = _sc_gather_ce(ids, tgt, w3)
    return _tc_reduce(sums, cts)
